# baseline (speedup 1.0000x reference)
import functools

import jax
import jax.numpy as jnp
from jax import lax
from jax.experimental import pallas as pl
from jax.experimental.pallas import tpu as pltpu
from jax.experimental.pallas import tpu_sc as plsc

NFIELD = 26
VOCAB = 100000
EMB = 16
BATCH = 4096
NCORE = 2
NSUB = 16
NWORK = NCORE * NSUB
BPW = BATCH // NWORK
NLOOK = NFIELD * BPW
FE = BPW * EMB
NRING = 8

_MESH = plsc.VectorSubcoreMesh(core_axis_name="c", subcore_axis_name="s")
_PARAMS = pltpu.CompilerParams(
    needs_layout_passes=False, use_tc_tiling_on_sc=False)


def _wid():
    return lax.axis_index("s") * NCORE + lax.axis_index("c")


NF_LO = 13
NF_HI = NFIELD - NF_LO


def _fm_emb_body(first, f0, nf, vidx_v, idx_v, ebuf_v, accs_v, accq_v,
                 init_v, emb_hbm, esem):
    if first:
        for d in range(EMB):
            for g in range(BPW // 16):
                o = d * BPW + g * 16
                row = init_v[pl.ds(o, 16)]
                accs_v[pl.ds(o, 16)] = row
                accq_v[pl.ds(o, 16)] = row * row

    def build_and_fire(f, p):
        fbase = pl.multiple_of((f0 + f) * BPW, BPW)
        ebase = f * (VOCAB * EMB)
        for d in range(EMB):
            dbase = ebase + d * VOCAB
            for c in range(BPW // 16):
                v16 = vidx_v[pl.ds(fbase + c * 16, 16)]
                idx_v[p, pl.ds(d * BPW + c * 16, 16)] = v16 + dbase
        for d in range(EMB):
            pltpu.async_copy(
                emb_hbm.at[idx_v.at[p, pl.ds(d * BPW, BPW)]],
                ebuf_v.at[p, pl.ds(d * BPW, BPW)], esem)

    for fp in range(NRING):
        build_and_fire(fp, fp)

    def field_body(f, carry):
        p = lax.bitwise_and(f, NRING - 1)
        for d in range(EMB):
            pltpu.make_async_copy(
                emb_hbm.at[pl.ds(0, BPW)],
                ebuf_v.at[p, pl.ds(d * BPW, BPW)], esem).wait()

        for d in range(EMB):
            for g in range(BPW // 16):
                o = d * BPW + g * 16
                ev = ebuf_v[p, pl.ds(o, 16)]
                accs_v[pl.ds(o, 16)] = accs_v[pl.ds(o, 16)] + ev
                accq_v[pl.ds(o, 16)] = accq_v[pl.ds(o, 16)] + ev * ev

        @pl.when(f + NRING < nf)
        def _():
            build_and_fire(f + NRING, p)

        return carry

    lax.fori_loop(0, nf, field_body, 0)


@functools.partial(
    pl.kernel,
    out_type=(jax.ShapeDtypeStruct((NWORK, FE), jnp.float32),
              jax.ShapeDtypeStruct((NWORK, FE), jnp.float32)),
    mesh=_MESH,
    compiler_params=_PARAMS,
    scratch_types=[
        pltpu.VMEM((NLOOK,), jnp.int32),
        pltpu.VMEM((NRING, FE), jnp.int32),
        pltpu.VMEM((NRING, FE), jnp.float32),
        pltpu.VMEM((FE,), jnp.float32),
        pltpu.VMEM((FE,), jnp.float32),
        pltpu.VMEM((FE,), jnp.float32),
        pltpu.SemaphoreType.DMA,
    ],
)
def _fm_emb_lo(vidx_hbm, emb_hbm, dprojt_hbm, accs_hbm, accq_hbm,
               vidx_v, idx_v, ebuf_v, accs_v, accq_v, dprojt_v, esem):
    wid = _wid()
    pltpu.sync_copy(vidx_hbm.at[wid], vidx_v)
    pltpu.sync_copy(dprojt_hbm.at[wid], dprojt_v)
    _fm_emb_body(True, 0, NF_LO, vidx_v, idx_v, ebuf_v, accs_v, accq_v,
                 dprojt_v, emb_hbm, esem)
    pltpu.sync_copy(accs_v, accs_hbm.at[wid])
    pltpu.sync_copy(accq_v, accq_hbm.at[wid])


@functools.partial(
    pl.kernel,
    out_type=jax.ShapeDtypeStruct((BATCH,), jnp.float32),
    mesh=_MESH,
    compiler_params=_PARAMS,
    scratch_types=[
        pltpu.VMEM((NLOOK,), jnp.int32),
        pltpu.VMEM((NRING, FE), jnp.int32),
        pltpu.VMEM((NRING, FE), jnp.float32),
        pltpu.VMEM((FE,), jnp.float32),
        pltpu.VMEM((FE,), jnp.float32),
        pltpu.VMEM((BPW,), jnp.float32),
        pltpu.SemaphoreType.DMA,
    ],
)
def _fm_emb_hi(vidx_hbm, emb_hbm, accs_hbm, accq_hbm, fm_hbm,
               vidx_v, idx_v, ebuf_v, accs_v, accq_v, fm_v, esem):
    wid = _wid()
    base = wid * BPW
    pltpu.sync_copy(vidx_hbm.at[wid], vidx_v)
    pltpu.sync_copy(accs_hbm.at[wid], accs_v)
    pltpu.sync_copy(accq_hbm.at[wid], accq_v)
    _fm_emb_body(False, NF_LO, NF_HI, vidx_v, idx_v, ebuf_v, accs_v,
                 accq_v, None, emb_hbm, esem)
    for g in range(BPW // 16):
        o = g * 16
        fm = jnp.zeros((16,), jnp.float32)
        for d in range(EMB):
            sv = accs_v[pl.ds(d * BPW + o, 16)]
            qv = accq_v[pl.ds(d * BPW + o, 16)]
            fm = fm + (sv * sv - qv)
        fm_v[pl.ds(o, 16)] = 0.5 * fm
    pltpu.sync_copy(fm_v, fm_hbm.at[pl.ds(base, BPW)])


def _fm_w_body(vidx_hbm, w_hbm, dlin_hbm, fm_hbm, out_hbm,
               vidx_v, wbuf_v, lin_v, fmin_v, out_v, wsem):
    wid = _wid()
    base = wid * BPW

    pltpu.sync_copy(vidx_hbm.at[wid], vidx_v)
    pltpu.sync_copy(dlin_hbm.at[pl.ds(base, BPW)], lin_v)
    pltpu.sync_copy(fm_hbm.at[pl.ds(base, BPW)], fmin_v)

    def fire(f, p):
        fbase = pl.multiple_of(f * BPW, BPW)
        pltpu.async_copy(
            w_hbm.at[f].at[vidx_v.at[pl.ds(fbase, BPW)]],
            wbuf_v.at[p], wsem)

    for f0 in range(NRING):
        fire(f0, f0)

    def field_body(f, carry):
        p = lax.bitwise_and(f, NRING - 1)
        pltpu.make_async_copy(
            w_hbm.at[0, pl.ds(0, BPW)], wbuf_v.at[p], wsem).wait()
        for g in range(BPW // 16):
            o = g * 16
            lin_v[pl.ds(o, 16)] = lin_v[pl.ds(o, 16)] + wbuf_v[p, pl.ds(o, 16)]

        @pl.when(f + NRING < NFIELD)
        def _():
            fire(f + NRING, p)

        return carry

    lax.fori_loop(0, NFIELD, field_body, 0)

    for g in range(BPW // 16):
        o = g * 16
        logit = lin_v[pl.ds(o, 16)] + fmin_v[pl.ds(o, 16)]
        out_v[pl.ds(o, 16)] = 1.0 / (1.0 + jnp.exp(-logit))

    pltpu.sync_copy(out_v, out_hbm.at[pl.ds(base, BPW)])


@functools.partial(
    pl.kernel,
    out_type=jax.ShapeDtypeStruct((BATCH,), jnp.float32),
    mesh=_MESH,
    compiler_params=_PARAMS,
    scratch_types=[
        pltpu.VMEM((NLOOK,), jnp.int32),
        pltpu.VMEM((NRING, BPW), jnp.float32),
        pltpu.VMEM((BPW,), jnp.float32),
        pltpu.VMEM((BPW,), jnp.float32),
        pltpu.VMEM((BPW,), jnp.float32),
        pltpu.SemaphoreType.DMA,
    ],
)
def _fm_w_call(vidx_hbm, w_hbm, dlin_hbm, fm_hbm, out_hbm,
               vidx_v, wbuf_v, lin_v, fmin_v, out_v, wsem):
    _fm_w_body(vidx_hbm, w_hbm, dlin_hbm, fm_hbm, out_hbm,
               vidx_v, wbuf_v, lin_v, fmin_v, out_v, wsem)


def kernel(sparse_features, dense_features, sparse_w, sparse_emb,
           dw_W, dw_b, de_W, de_b, bias):
    vidx = sparse_features.astype(jnp.int32).reshape(
        NWORK, BPW, NFIELD).transpose(0, 2, 1).reshape(NWORK, NLOOK)
    emb_lo = sparse_emb[:NF_LO].transpose(0, 2, 1).reshape(
        NF_LO * EMB * VOCAB)
    emb_hi = sparse_emb[NF_LO:].transpose(0, 2, 1).reshape(
        NF_HI * EMB * VOCAB)
    w2d = jnp.pad(lax.squeeze(sparse_w, (2,)), ((0, 0), (0, 96)))
    dprojt = (dense_features @ de_W + de_b).reshape(
        NWORK, BPW, EMB).transpose(0, 2, 1).reshape(NWORK, FE)
    dlin = (dense_features @ dw_W)[:, 0] + dw_b[0] + bias[0]
    accs, accq = _fm_emb_lo(vidx, emb_lo, dprojt)
    fm = _fm_emb_hi(vidx, emb_hi, accs, accq)
    return _fm_w_call(vidx, w2d, dlin, fm)


# speedup vs baseline: 1.2369x; 1.2369x over previous
import functools

import jax
import jax.numpy as jnp
from jax import lax
from jax.experimental import pallas as pl
from jax.experimental.pallas import tpu as pltpu
from jax.experimental.pallas import tpu_sc as plsc

NFIELD = 26
VOCAB = 100000
EMB = 16
BATCH = 4096
NCORE = 2
NSUB = 16
NWORK = NCORE * NSUB
BPW = BATCH // NWORK
NLOOK = NFIELD * BPW
FE = BPW * EMB
NRING = 8

_MESH = plsc.VectorSubcoreMesh(core_axis_name="c", subcore_axis_name="s")
_PARAMS = pltpu.CompilerParams(
    needs_layout_passes=False, use_tc_tiling_on_sc=False)


def _wid():
    return lax.axis_index("s") * NCORE + lax.axis_index("c")


def _fm_emb_body(vidx_hbm, emb_hbm, dprojt_hbm, fm_hbm,
                 vidx_v, idx_v, ebuf_v, accs_v, accq_v, dprojt_v, fm_v,
                 esem):
    wid = _wid()
    base = wid * BPW

    pltpu.sync_copy(vidx_hbm.at[wid], vidx_v)
    pltpu.sync_copy(dprojt_hbm.at[wid], dprojt_v)

    for d in range(EMB):
        for g in range(BPW // 16):
            o = d * BPW + g * 16
            row = dprojt_v[pl.ds(o, 16)]
            accs_v[pl.ds(o, 16)] = row
            accq_v[pl.ds(o, 16)] = row * row

    def build_and_fire(f, p):
        fbase = pl.multiple_of(f * BPW, BPW)
        ebase = f * (VOCAB * EMB)
        for d in range(EMB):
            dbase = ebase + d * VOCAB
            for c in range(BPW // 16):
                v16 = vidx_v[pl.ds(fbase + c * 16, 16)]
                idx_v[p, pl.ds(d * BPW + c * 16, 16)] = v16 + dbase
        for d in range(EMB):
            pltpu.async_copy(
                emb_hbm.at[idx_v.at[p, pl.ds(d * BPW, BPW)]],
                ebuf_v.at[p, pl.ds(d * BPW, BPW)], esem)

    for f0 in range(NRING):
        build_and_fire(f0, f0)

    def field_body(f, carry):
        p = lax.bitwise_and(f, NRING - 1)
        for d in range(EMB):
            pltpu.make_async_copy(
                emb_hbm.at[pl.ds(0, BPW)],
                ebuf_v.at[p, pl.ds(d * BPW, BPW)], esem).wait()

        for d in range(EMB):
            for g in range(BPW // 16):
                o = d * BPW + g * 16
                ev = ebuf_v[p, pl.ds(o, 16)]
                accs_v[pl.ds(o, 16)] = accs_v[pl.ds(o, 16)] + ev
                accq_v[pl.ds(o, 16)] = accq_v[pl.ds(o, 16)] + ev * ev

        @pl.when(f + NRING < NFIELD)
        def _():
            build_and_fire(f + NRING, p)

        return carry

    lax.fori_loop(0, NFIELD, field_body, 0)

    for g in range(BPW // 16):
        o = g * 16
        fm = jnp.zeros((16,), jnp.float32)
        for d in range(EMB):
            sv = accs_v[pl.ds(d * BPW + o, 16)]
            qv = accq_v[pl.ds(d * BPW + o, 16)]
            fm = fm + (sv * sv - qv)
        fm_v[pl.ds(o, 16)] = 0.5 * fm

    pltpu.sync_copy(fm_v, fm_hbm.at[pl.ds(base, BPW)])


@functools.partial(
    pl.kernel,
    out_type=jax.ShapeDtypeStruct((BATCH,), jnp.float32),
    mesh=_MESH,
    compiler_params=_PARAMS,
    scratch_types=[
        pltpu.VMEM((NLOOK,), jnp.int32),
        pltpu.VMEM((NRING, FE), jnp.int32),
        pltpu.VMEM((NRING, FE), jnp.float32),
        pltpu.VMEM((FE,), jnp.float32),
        pltpu.VMEM((FE,), jnp.float32),
        pltpu.VMEM((FE,), jnp.float32),
        pltpu.VMEM((BPW,), jnp.float32),
        pltpu.SemaphoreType.DMA,
    ],
)
def _fm_emb_call(vidx_hbm, emb_hbm, dprojt_hbm, fm_hbm,
                 vidx_v, idx_v, ebuf_v, accs_v, accq_v, dprojt_v, fm_v,
                 esem):
    _fm_emb_body(vidx_hbm, emb_hbm, dprojt_hbm, fm_hbm,
                 vidx_v, idx_v, ebuf_v, accs_v, accq_v, dprojt_v, fm_v,
                 esem)


def _fm_w_body(vidx_hbm, w_hbm, dlin_hbm, fm_hbm, out_hbm,
               vidx_v, wbuf_v, lin_v, fmin_v, out_v, wsem):
    wid = _wid()
    base = wid * BPW

    pltpu.sync_copy(vidx_hbm.at[wid], vidx_v)
    pltpu.sync_copy(dlin_hbm.at[pl.ds(base, BPW)], lin_v)
    pltpu.sync_copy(fm_hbm.at[pl.ds(base, BPW)], fmin_v)

    def fire(f, p):
        fbase = pl.multiple_of(f * BPW, BPW)
        pltpu.async_copy(
            w_hbm.at[f].at[vidx_v.at[pl.ds(fbase, BPW)]],
            wbuf_v.at[p], wsem)

    for f0 in range(NRING):
        fire(f0, f0)

    def field_body(f, carry):
        p = lax.bitwise_and(f, NRING - 1)
        pltpu.make_async_copy(
            w_hbm.at[0, pl.ds(0, BPW)], wbuf_v.at[p], wsem).wait()
        for g in range(BPW // 16):
            o = g * 16
            lin_v[pl.ds(o, 16)] = lin_v[pl.ds(o, 16)] + wbuf_v[p, pl.ds(o, 16)]

        @pl.when(f + NRING < NFIELD)
        def _():
            fire(f + NRING, p)

        return carry

    lax.fori_loop(0, NFIELD, field_body, 0)

    for g in range(BPW // 16):
        o = g * 16
        logit = lin_v[pl.ds(o, 16)] + fmin_v[pl.ds(o, 16)]
        out_v[pl.ds(o, 16)] = 1.0 / (1.0 + jnp.exp(-logit))

    pltpu.sync_copy(out_v, out_hbm.at[pl.ds(base, BPW)])


@functools.partial(
    pl.kernel,
    out_type=jax.ShapeDtypeStruct((BATCH,), jnp.float32),
    mesh=_MESH,
    compiler_params=_PARAMS,
    scratch_types=[
        pltpu.VMEM((NLOOK,), jnp.int32),
        pltpu.VMEM((NRING, BPW), jnp.float32),
        pltpu.VMEM((BPW,), jnp.float32),
        pltpu.VMEM((BPW,), jnp.float32),
        pltpu.VMEM((BPW,), jnp.float32),
        pltpu.SemaphoreType.DMA,
    ],
)
def _fm_w_call(vidx_hbm, w_hbm, dlin_hbm, fm_hbm, out_hbm,
               vidx_v, wbuf_v, lin_v, fmin_v, out_v, wsem):
    _fm_w_body(vidx_hbm, w_hbm, dlin_hbm, fm_hbm, out_hbm,
               vidx_v, wbuf_v, lin_v, fmin_v, out_v, wsem)


def kernel(sparse_features, dense_features, sparse_w, sparse_emb,
           dw_W, dw_b, de_W, de_b, bias):
    vidx = sparse_features.astype(jnp.int32).reshape(
        NWORK, BPW, NFIELD).transpose(0, 2, 1).reshape(NWORK, NLOOK)
    emb_dm = sparse_emb.transpose(0, 2, 1).reshape(NFIELD * EMB * VOCAB)
    w2d = jnp.pad(lax.squeeze(sparse_w, (2,)), ((0, 0), (0, 96)))
    dprojt = (dense_features @ de_W + de_b).reshape(
        NWORK, BPW, EMB).transpose(0, 2, 1).reshape(NWORK, FE)
    dlin = (dense_features @ dw_W)[:, 0] + dw_b[0] + bias[0]
    fm = _fm_emb_call(vidx, emb_dm, dprojt)
    return _fm_w_call(vidx, w2d, dlin, fm)
